# Initial kernel scaffold; baseline (speedup 1.0000x reference)
import jax
import jax.numpy as jnp
from jax.experimental import pallas as pl


def kernel(playlist_weight, track_weight, edge_index, edge_weight):
    raise NotImplementedError("write your pallas kernel here")


# speedup vs baseline: 6.6019x; 6.6019x over previous
import functools

import jax
import jax.numpy as jnp
from jax import lax
from jax.experimental import pallas as pl
from jax.experimental.pallas import tpu as pltpu
from jax.experimental.pallas import tpu_sc as plsc

_NUM_PLAYLISTS = 2000
_NUM_TRACKS = 8000
_D = 128
_N = _NUM_PLAYLISTS + _NUM_TRACKS
_N_PAD = 10240
_E = 320000
_NLAYERS = 3

_CH = 128
_NWORKERS = 32
_NCH_PER_TILE = 80
_E_PAD = _NWORKERS * _NCH_PER_TILE * _CH
_NCH_TOTAL = _E_PAD // _CH
_ROWS_PER_TILE = _N_PAD // 16
_LANES = 16

_mesh = plsc.VectorSubcoreMesh(core_axis_name="c", subcore_axis_name="s")


@functools.partial(
    pl.kernel,
    out_type=[
        jax.ShapeDtypeStruct((_N_PAD, _D), jnp.float32),
        jax.ShapeDtypeStruct((_N_PAD, _D), jnp.float32),
    ],
    mesh=_mesh,
    scratch_types=[
        pltpu.VMEM((_NCH_PER_TILE, _CH), jnp.int32),
        pltpu.VMEM((_NCH_PER_TILE, _CH), jnp.int32),
        pltpu.VMEM((_NCH_PER_TILE, _CH), jnp.float32),
        pltpu.VMEM((_CH, _D), jnp.float32),
        pltpu.VMEM_SHARED((_N_PAD, _D), jnp.float32),
        pltpu.SemaphoreType.DMA,
    ],
)
def _sc_layer(emb, src2d, dst2d, w2d, out0, out1,
              srcv, dstv, wv, rows, acc, sem):
    c = lax.axis_index("c")
    s = lax.axis_index("s")
    wid = s * 2 + c
    cbase = wid * _NCH_PER_TILE
    pltpu.sync_copy(src2d.at[pl.ds(cbase, _NCH_PER_TILE)], srcv)
    pltpu.sync_copy(dst2d.at[pl.ds(cbase, _NCH_PER_TILE)], dstv)
    pltpu.sync_copy(w2d.at[pl.ds(cbase, _NCH_PER_TILE)], wv)

    def _zero_row(i, _):
        for d in range(_D // _LANES):
            rows[i, pl.ds(d * _LANES, _LANES)] = jnp.zeros((_LANES,), jnp.float32)
        return 0
    lax.fori_loop(0, _CH, _zero_row, 0)
    rbase = s * _ROWS_PER_TILE
    for k in range(_ROWS_PER_TILE // _CH):
        pltpu.sync_copy(rows, acc.at[pl.ds(rbase + k * _CH, _CH)])
    plsc.subcore_barrier()

    def _chunk(g, _):
        pltpu.async_copy(emb.at[srcv.at[g]], rows, sem).wait()

        def _scale(eg, _):
            wvec = wv[g, pl.ds(eg * _LANES, _LANES)]
            for j in range(_LANES):
                wsc = wvec[j]
                e = eg * _LANES + j
                for d in range(_D // _LANES):
                    sl = pl.ds(d * _LANES, _LANES)
                    rows[e, sl] = rows[e, sl] * wsc
            return 0
        lax.fori_loop(0, _CH // _LANES, _scale, 0)
        pltpu.sync_copy(rows, acc.at[dstv.at[g]], add=True)
        return 0
    lax.fori_loop(0, _NCH_PER_TILE, _chunk, 0)
    plsc.subcore_barrier()

    for k in range(_ROWS_PER_TILE // _CH):
        sl = pl.ds(rbase + k * _CH, _CH)

        @pl.when(c == 0)
        def _():
            pltpu.sync_copy(acc.at[sl], out0.at[sl])

        @pl.when(c == 1)
        def _():
            pltpu.sync_copy(acc.at[sl], out1.at[sl])


def _combine_body(pa_ref, pb_ref, run_ref, emb_ref, runo_ref):
    sm = pa_ref[...] + pb_ref[...]
    emb_ref[...] = sm
    runo_ref[...] = run_ref[...] + sm


def _final_body(pa_ref, pb_ref, run_ref, out_ref):
    out_ref[...] = (run_ref[...] + pa_ref[...] + pb_ref[...]) * 0.25


_bs = pl.BlockSpec((1024, _D), lambda i: (i, 0))
_sds = jax.ShapeDtypeStruct((_N_PAD, _D), jnp.float32)

_combine = pl.pallas_call(
    _combine_body, grid=(10,), in_specs=[_bs, _bs, _bs],
    out_specs=[_bs, _bs], out_shape=[_sds, _sds])

_final = pl.pallas_call(
    _final_body, grid=(10,), in_specs=[_bs, _bs, _bs],
    out_specs=_bs, out_shape=_sds)


def kernel(playlist_weight, track_weight, edge_index, edge_weight):
    emb0 = jnp.concatenate([playlist_weight, track_weight], axis=0)
    emb0 = jnp.pad(emb0, ((0, _N_PAD - _N), (0, 0)))
    src = edge_index[0].astype(jnp.int32)
    dst = edge_index[1].astype(jnp.int32)
    w = edge_weight.astype(jnp.float32)
    pad = _E_PAD - _E
    fill = _N + jnp.arange(pad, dtype=jnp.int32) % (_N_PAD - _N)
    src2d = jnp.concatenate([src, fill]).reshape(_NCH_TOTAL, _CH)
    dst2d = jnp.concatenate([dst, fill]).reshape(_NCH_TOTAL, _CH)
    w2d = jnp.concatenate([w, jnp.zeros((pad,), jnp.float32)]).reshape(_NCH_TOTAL, _CH)

    emb = emb0
    run = emb0
    final = None
    for layer in range(_NLAYERS):
        pa, pb = _sc_layer(emb, src2d, dst2d, w2d)
        if layer < _NLAYERS - 1:
            emb, run = _combine(pa, pb, run)
        else:
            final = _final(pa, pb, run)
    return final[:_NUM_PLAYLISTS], final[_NUM_PLAYLISTS:_N]


# speedup vs baseline: 9.0733x; 1.3744x over previous
import functools

import jax
import jax.numpy as jnp
from jax import lax
from jax.experimental import pallas as pl
from jax.experimental.pallas import tpu as pltpu
from jax.experimental.pallas import tpu_sc as plsc

_NUM_PLAYLISTS = 2000
_NUM_TRACKS = 8000
_D = 128
_N = _NUM_PLAYLISTS + _NUM_TRACKS
_N_PAD = 10240
_E = 320000
_NLAYERS = 3

_CH = 128
_NWORKERS = 32
_NCH_PER_TILE = 80
_E_PAD = _NWORKERS * _NCH_PER_TILE * _CH
_NCH_TOTAL = _E_PAD // _CH
_ROWS_PER_TILE = _N_PAD // 16
_LANES = 16
_HALF = _NCH_PER_TILE // 2
_NPAIR = _HALF // 2

_mesh = plsc.VectorSubcoreMesh(core_axis_name="c", subcore_axis_name="s")


@functools.partial(
    pl.kernel,
    out_type=[
        jax.ShapeDtypeStruct((_N_PAD, _D), jnp.float32),
        jax.ShapeDtypeStruct((_N_PAD, _D), jnp.float32),
    ],
    mesh=_mesh,
    scratch_types=[
        pltpu.VMEM((_HALF, _CH), jnp.int32),
        pltpu.VMEM((_HALF, _CH), jnp.int32),
        pltpu.VMEM((_HALF, _CH), jnp.float32),
        pltpu.VMEM((_CH, _D), jnp.float32),
        pltpu.VMEM((_CH, _D), jnp.float32),
        pltpu.VMEM_SHARED((_N_PAD, _D), jnp.float32),
        pltpu.SemaphoreType.DMA,
        pltpu.SemaphoreType.DMA,
        pltpu.SemaphoreType.DMA,
        pltpu.SemaphoreType.DMA,
    ],
)
def _sc_layer(emb, src2d, dst2d, w2d, out0, out1,
              srcv, dstv, wv, rows0, rows1, acc, gsem0, gsem1, ssem0, ssem1):
    rows = rows0
    c = lax.axis_index("c")
    s = lax.axis_index("s")
    wid = s * 2 + c

    def _zero_row(i, _):
        for d in range(_D // _LANES):
            rows[i, pl.ds(d * _LANES, _LANES)] = jnp.zeros((_LANES,), jnp.float32)
        return 0
    lax.fori_loop(0, _CH, _zero_row, 0)
    rbase = s * _ROWS_PER_TILE
    for k in range(_ROWS_PER_TILE // _CH):
        pltpu.sync_copy(rows, acc.at[pl.ds(rbase + k * _CH, _CH)])
    plsc.subcore_barrier()

    def _scale(rows_b, g):
        def body(eg, _):
            wvec = wv[g, pl.ds(eg * _LANES, _LANES)]
            for j in range(_LANES):
                wsc = wvec[j]
                e = eg * _LANES + j
                for d in range(_D // _LANES):
                    sl = pl.ds(d * _LANES, _LANES)
                    rows_b[e, sl] = rows_b[e, sl] * wsc
            return 0
        lax.fori_loop(0, _CH // _LANES, body, 0)

    def _pipe(i, _):
        g0 = i * 2
        g1 = g0 + 1
        pltpu.make_async_copy(emb.at[srcv.at[g0]], rows0, gsem0).wait()

        @pl.when(i > 0)
        def _():
            pltpu.make_async_copy(rows1, acc.at[dstv.at[g1 - 2]], ssem1).wait()

        pltpu.async_copy(emb.at[srcv.at[g1]], rows1, gsem1)
        _scale(rows0, g0)
        pltpu.async_copy(rows0, acc.at[dstv.at[g0]], ssem0, add=True)

        pltpu.make_async_copy(emb.at[srcv.at[g1]], rows1, gsem1).wait()
        _scale(rows1, g1)

        @pl.when(i < _NPAIR - 1)
        def _():
            pltpu.make_async_copy(rows0, acc.at[dstv.at[g0]], ssem0).wait()
            pltpu.async_copy(emb.at[srcv.at[g0 + 2]], rows0, gsem0)

        pltpu.async_copy(rows1, acc.at[dstv.at[g1]], ssem1, add=True)
        return 0

    for half in range(_NCH_PER_TILE // _HALF):
        cbase = wid * _NCH_PER_TILE + half * _HALF
        pltpu.sync_copy(src2d.at[pl.ds(cbase, _HALF)], srcv)
        pltpu.sync_copy(dst2d.at[pl.ds(cbase, _HALF)], dstv)
        pltpu.sync_copy(w2d.at[pl.ds(cbase, _HALF)], wv)
        pltpu.async_copy(emb.at[srcv.at[0]], rows0, gsem0)
        lax.fori_loop(0, _NPAIR, _pipe, 0)
        pltpu.make_async_copy(rows0, acc.at[dstv.at[_HALF - 2]], ssem0).wait()
        pltpu.make_async_copy(rows1, acc.at[dstv.at[_HALF - 1]], ssem1).wait()
    plsc.subcore_barrier()

    for k in range(_ROWS_PER_TILE // _CH):
        sl = pl.ds(rbase + k * _CH, _CH)

        @pl.when(c == 0)
        def _():
            pltpu.sync_copy(acc.at[sl], out0.at[sl])

        @pl.when(c == 1)
        def _():
            pltpu.sync_copy(acc.at[sl], out1.at[sl])


def _combine_body(pa_ref, pb_ref, run_ref, emb_ref, runo_ref):
    sm = pa_ref[...] + pb_ref[...]
    emb_ref[...] = sm
    runo_ref[...] = run_ref[...] + sm


def _final_body(pa_ref, pb_ref, run_ref, out_ref):
    out_ref[...] = (run_ref[...] + pa_ref[...] + pb_ref[...]) * 0.25


_bs = pl.BlockSpec((1024, _D), lambda i: (i, 0))
_sds = jax.ShapeDtypeStruct((_N_PAD, _D), jnp.float32)

_combine = pl.pallas_call(
    _combine_body, grid=(10,), in_specs=[_bs, _bs, _bs],
    out_specs=[_bs, _bs], out_shape=[_sds, _sds])

_final = pl.pallas_call(
    _final_body, grid=(10,), in_specs=[_bs, _bs, _bs],
    out_specs=_bs, out_shape=_sds)


def kernel(playlist_weight, track_weight, edge_index, edge_weight):
    emb0 = jnp.concatenate([playlist_weight, track_weight], axis=0)
    emb0 = jnp.pad(emb0, ((0, _N_PAD - _N), (0, 0)))
    src = edge_index[0].astype(jnp.int32)
    dst = edge_index[1].astype(jnp.int32)
    w = edge_weight.astype(jnp.float32)
    pad = _E_PAD - _E
    fill = _N + jnp.arange(pad, dtype=jnp.int32) % (_N_PAD - _N)
    src2d = jnp.concatenate([src, fill]).reshape(_NCH_TOTAL, _CH)
    dst2d = jnp.concatenate([dst, fill]).reshape(_NCH_TOTAL, _CH)
    w2d = jnp.concatenate([w, jnp.zeros((pad,), jnp.float32)]).reshape(_NCH_TOTAL, _CH)

    emb = emb0
    run = emb0
    final = None
    for layer in range(_NLAYERS):
        pa, pb = _sc_layer(emb, src2d, dst2d, w2d)
        if layer < _NLAYERS - 1:
            emb, run = _combine(pa, pb, run)
        else:
            final = _final(pa, pb, run)
    return final[:_NUM_PLAYLISTS], final[_NUM_PLAYLISTS:_N]


# speedup vs baseline: 10.7617x; 1.1861x over previous
import functools

import jax
import jax.numpy as jnp
from jax import lax
from jax.experimental import pallas as pl
from jax.experimental.pallas import tpu as pltpu
from jax.experimental.pallas import tpu_sc as plsc

_NUM_PLAYLISTS = 2000
_NUM_TRACKS = 8000
_D = 128
_N = _NUM_PLAYLISTS + _NUM_TRACKS
_N_PAD = 10240
_E = 320000
_NLAYERS = 3

_CH = 64
_NWORKERS = 32
_NCHT = 160
_E_PAD = _NWORKERS * _NCHT * _CH
_ROWS_PER_TILE = _N_PAD // 16
_LANES = 16
_NBUF = 4

_mesh = plsc.VectorSubcoreMesh(core_axis_name="c", subcore_axis_name="s")

_scratch = (
    [pltpu.VMEM((_CH,), jnp.int32) for _ in range(_NBUF)]
    + [pltpu.VMEM((_CH,), jnp.int32) for _ in range(_NBUF)]
    + [pltpu.VMEM((_CH,), jnp.float32) for _ in range(_NBUF)]
    + [pltpu.VMEM((_CH, _D), jnp.float32) for _ in range(_NBUF)]
    + [pltpu.VMEM_SHARED((_N_PAD, _D), jnp.float32)]
    + [pltpu.SemaphoreType.DMA for _ in range(3 * _NBUF)]
)


@functools.partial(
    pl.kernel,
    out_type=[
        jax.ShapeDtypeStruct((_N_PAD, _D), jnp.float32),
        jax.ShapeDtypeStruct((_N_PAD, _D), jnp.float32),
    ],
    mesh=_mesh,
    scratch_types=_scratch,
)
def _sc_layer(emb, src1, dst1, w1, out0, out1, *refs):
    srcb = refs[0:_NBUF]
    dstb = refs[_NBUF:2 * _NBUF]
    wb = refs[2 * _NBUF:3 * _NBUF]
    rows = refs[3 * _NBUF:4 * _NBUF]
    acc = refs[4 * _NBUF]
    esem = refs[4 * _NBUF + 1:4 * _NBUF + 1 + _NBUF]
    gsem = refs[4 * _NBUF + 1 + _NBUF:4 * _NBUF + 1 + 2 * _NBUF]
    ssem = refs[4 * _NBUF + 1 + 2 * _NBUF:4 * _NBUF + 1 + 3 * _NBUF]

    c = lax.axis_index("c")
    s = lax.axis_index("s")
    wid = s * 2 + c
    ebase = wid * _NCHT * _CH

    def _zero_row(i, _):
        for d in range(_D // _LANES):
            rows[0][i, pl.ds(d * _LANES, _LANES)] = jnp.zeros((_LANES,), jnp.float32)
        return 0
    lax.fori_loop(0, _CH, _zero_row, 0)
    rbase = s * _ROWS_PER_TILE
    for k in range(_ROWS_PER_TILE // _CH):
        pltpu.sync_copy(rows[0], acc.at[pl.ds(rbase + k * _CH, _CH)])
    plsc.subcore_barrier()

    def _efetch(g, b):
        cb = ebase + g * _CH
        pltpu.async_copy(src1.at[pl.ds(cb, _CH)], srcb[b], esem[b])
        pltpu.async_copy(dst1.at[pl.ds(cb, _CH)], dstb[b], esem[b])
        pltpu.async_copy(w1.at[pl.ds(cb, _CH)], wb[b], esem[b])

    def _efwait(g, b):
        cb = ebase + g * _CH
        pltpu.make_async_copy(src1.at[pl.ds(cb, _CH)], srcb[b], esem[b]).wait()
        pltpu.make_async_copy(dst1.at[pl.ds(cb, _CH)], dstb[b], esem[b]).wait()
        pltpu.make_async_copy(w1.at[pl.ds(cb, _CH)], wb[b], esem[b]).wait()

    def _scale(b):
        def body(eg, _):
            wvec = wb[b][pl.ds(eg * _LANES, _LANES)]
            for j in range(_LANES):
                wsc = wvec[j]
                e = eg * _LANES + j
                for d in range(_D // _LANES):
                    sl = pl.ds(d * _LANES, _LANES)
                    rows[b][e, sl] = rows[b][e, sl] * wsc
            return 0
        lax.fori_loop(0, _CH // _LANES, body, 0)

    _efetch(0, 0)
    _efetch(1, 1)
    _efetch(2, 2)
    _efwait(0, 0)
    pltpu.async_copy(emb.at[srcb[0]], rows[0], gsem[0])
    _efwait(1, 1)
    pltpu.async_copy(emb.at[srcb[1]], rows[1], gsem[1])

    def _iter(i, _):
        for b in range(_NBUF):
            g = i * _NBUF + b
            pltpu.make_async_copy(emb.at[srcb[b]], rows[b], gsem[b]).wait()
            _scale(b)
            pltpu.async_copy(rows[b], acc.at[dstb[b]], ssem[b], add=True)

            bm2 = (b - 2) % _NBUF

            @pl.when(g >= 2)
            def _():
                pltpu.make_async_copy(rows[bm2], acc.at[dstb[bm2]], ssem[bm2]).wait()

            bp3 = (b + 3) % _NBUF

            @pl.when(g + 3 < _NCHT)
            def _():
                _efetch(g + 3, bp3)

            bp2 = (b + 2) % _NBUF

            @pl.when(g + 2 < _NCHT)
            def _():
                _efwait(g + 2, bp2)
                pltpu.async_copy(emb.at[srcb[bp2]], rows[bp2], gsem[bp2])
        return 0
    lax.fori_loop(0, _NCHT // _NBUF, _iter, 0)
    pltpu.make_async_copy(rows[2], acc.at[dstb[2]], ssem[2]).wait()
    pltpu.make_async_copy(rows[3], acc.at[dstb[3]], ssem[3]).wait()
    plsc.subcore_barrier()

    for k in range(_ROWS_PER_TILE // 128):
        sl = pl.ds(rbase + k * 128, 128)

        @pl.when(c == 0)
        def _():
            pltpu.sync_copy(acc.at[sl], out0.at[sl])

        @pl.when(c == 1)
        def _():
            pltpu.sync_copy(acc.at[sl], out1.at[sl])


def _combine_body(pa_ref, pb_ref, run_ref, emb_ref, runo_ref):
    sm = pa_ref[...] + pb_ref[...]
    emb_ref[...] = sm
    runo_ref[...] = run_ref[...] + sm


def _final_body(pa_ref, pb_ref, run_ref, out_ref):
    out_ref[...] = (run_ref[...] + pa_ref[...] + pb_ref[...]) * 0.25


_bs = pl.BlockSpec((1024, _D), lambda i: (i, 0))
_sds = jax.ShapeDtypeStruct((_N_PAD, _D), jnp.float32)

_combine = pl.pallas_call(
    _combine_body, grid=(10,), in_specs=[_bs, _bs, _bs],
    out_specs=[_bs, _bs], out_shape=[_sds, _sds])

_final = pl.pallas_call(
    _final_body, grid=(10,), in_specs=[_bs, _bs, _bs],
    out_specs=_bs, out_shape=_sds)


def kernel(playlist_weight, track_weight, edge_index, edge_weight):
    emb0 = jnp.concatenate([playlist_weight, track_weight], axis=0)
    emb0 = jnp.pad(emb0, ((0, _N_PAD - _N), (0, 0)))
    src = edge_index[0].astype(jnp.int32)
    dst = edge_index[1].astype(jnp.int32)
    w = edge_weight.astype(jnp.float32)
    pad = _E_PAD - _E
    fill = _N + jnp.arange(pad, dtype=jnp.int32) % (_N_PAD - _N)
    src1 = jnp.concatenate([src, fill])
    dst1 = jnp.concatenate([dst, fill])
    w1 = jnp.concatenate([w, jnp.zeros((pad,), jnp.float32)])

    emb = emb0
    run = emb0
    final = None
    for layer in range(_NLAYERS):
        pa, pb = _sc_layer(emb, src1, dst1, w1)
        if layer < _NLAYERS - 1:
            emb, run = _combine(pa, pb, run)
        else:
            final = _final(pa, pb, run)
    return final[:_NUM_PLAYLISTS], final[_NUM_PLAYLISTS:_N]


# speedup vs baseline: 11.3167x; 1.0516x over previous
import functools

import jax
import jax.numpy as jnp
from jax import lax
from jax.experimental import pallas as pl
from jax.experimental.pallas import tpu as pltpu
from jax.experimental.pallas import tpu_sc as plsc

_NUM_PLAYLISTS = 2000
_NUM_TRACKS = 8000
_D = 128
_N = _NUM_PLAYLISTS + _NUM_TRACKS
_N_PAD = 10240
_E = 320000
_NLAYERS = 3

_CH = 64
_NWORKERS = 32
_NCHT = 160
_E_PAD = _NWORKERS * _NCHT * _CH
_ROWS_PER_TILE = _N_PAD // 16
_LANES = 16
_NBUF = 5

_mesh = plsc.VectorSubcoreMesh(core_axis_name="c", subcore_axis_name="s")

_scratch = (
    [pltpu.VMEM((_CH,), jnp.int32) for _ in range(_NBUF)]
    + [pltpu.VMEM((_CH,), jnp.int32) for _ in range(_NBUF)]
    + [pltpu.VMEM((_CH,), jnp.float32) for _ in range(_NBUF)]
    + [pltpu.VMEM((_CH, _D), jnp.float32) for _ in range(_NBUF)]
    + [pltpu.VMEM_SHARED((_N_PAD, _D), jnp.float32)]
    + [pltpu.SemaphoreType.DMA for _ in range(4 * _NBUF)]
)


@functools.partial(
    pl.kernel,
    out_type=[
        jax.ShapeDtypeStruct((_N_PAD, _D), jnp.float32),
        jax.ShapeDtypeStruct((_N_PAD, _D), jnp.float32),
    ],
    mesh=_mesh,
    scratch_types=_scratch,
)
def _sc_layer(emb, src1, dst1, w1, out0, out1, *refs):
    srcb = refs[0:_NBUF]
    dstb = refs[_NBUF:2 * _NBUF]
    wb = refs[2 * _NBUF:3 * _NBUF]
    rows = refs[3 * _NBUF:4 * _NBUF]
    acc = refs[4 * _NBUF]
    esemS = refs[4 * _NBUF + 1:4 * _NBUF + 1 + _NBUF]
    esemD = refs[4 * _NBUF + 1 + _NBUF:4 * _NBUF + 1 + 2 * _NBUF]
    gsem = refs[4 * _NBUF + 1 + 2 * _NBUF:4 * _NBUF + 1 + 3 * _NBUF]
    ssem = refs[4 * _NBUF + 1 + 3 * _NBUF:4 * _NBUF + 1 + 4 * _NBUF]

    c = lax.axis_index("c")
    s = lax.axis_index("s")
    wid = s * 2 + c
    ebase = wid * _NCHT * _CH

    def _zero_row(i, _):
        for d in range(_D // _LANES):
            rows[0][i, pl.ds(d * _LANES, _LANES)] = jnp.zeros((_LANES,), jnp.float32)
        return 0
    lax.fori_loop(0, _CH, _zero_row, 0)
    rbase = s * _ROWS_PER_TILE
    for k in range(_ROWS_PER_TILE // _CH):
        pltpu.sync_copy(rows[0], acc.at[pl.ds(rbase + k * _CH, _CH)])
    plsc.subcore_barrier()

    def _scale(b):
        def body(eg, _):
            wvec = wb[b][pl.ds(eg * _LANES, _LANES)]
            for j in range(_LANES):
                wsc = wvec[j]
                e = eg * _LANES + j
                for d in range(_D // _LANES):
                    sl = pl.ds(d * _LANES, _LANES)
                    rows[b][e, sl] = rows[b][e, sl] * wsc
            return 0
        lax.fori_loop(0, _CH // _LANES, body, 0)

    def _ef_srcw(g, b):
        cb = ebase + g * _CH
        pltpu.async_copy(src1.at[pl.ds(cb, _CH)], srcb[b], esemS[b])
        pltpu.async_copy(w1.at[pl.ds(cb, _CH)], wb[b], esemS[b])

    def _efwait_srcw(g, b):
        cb = ebase + g * _CH
        pltpu.make_async_copy(src1.at[pl.ds(cb, _CH)], srcb[b], esemS[b]).wait()
        pltpu.make_async_copy(w1.at[pl.ds(cb, _CH)], wb[b], esemS[b]).wait()

    def _ef_dst(g, b):
        cb = ebase + g * _CH
        pltpu.async_copy(dst1.at[pl.ds(cb, _CH)], dstb[b], esemD[b])

    def _efwait_dst(g, b):
        cb = ebase + g * _CH
        pltpu.make_async_copy(dst1.at[pl.ds(cb, _CH)], dstb[b], esemD[b]).wait()

    for g in range(4):
        _ef_srcw(g, g)
    for g in range(3):
        _ef_dst(g, g)
    for g in range(3):
        _efwait_srcw(g, g)
        pltpu.async_copy(emb.at[srcb[g]], rows[g], gsem[g])

    def _iter(i, _):
        for b in range(_NBUF):
            g = i * _NBUF + b
            pltpu.make_async_copy(emb.at[srcb[b]], rows[b], gsem[b]).wait()
            _scale(b)
            _efwait_dst(g, b)
            pltpu.async_copy(rows[b], acc.at[dstb[b]], ssem[b], add=True)

            bm2 = (b - 2) % _NBUF

            @pl.when(g >= 2)
            def _():
                pltpu.make_async_copy(rows[bm2], acc.at[dstb[bm2]], ssem[bm2]).wait()

            bp3 = (b + 3) % _NBUF
            bp4 = (b + 4) % _NBUF

            @pl.when(g + 3 < _NCHT)
            def _():
                _ef_dst(g + 3, bp3)

            @pl.when(g + 4 < _NCHT)
            def _():
                _ef_srcw(g + 4, bp4)

            @pl.when(g + 3 < _NCHT)
            def _():
                _efwait_srcw(g + 3, bp3)
                pltpu.async_copy(emb.at[srcb[bp3]], rows[bp3], gsem[bp3])
        return 0
    lax.fori_loop(0, _NCHT // _NBUF, _iter, 0)
    pltpu.make_async_copy(rows[(_NCHT - 2) % _NBUF], acc.at[dstb[(_NCHT - 2) % _NBUF]], ssem[(_NCHT - 2) % _NBUF]).wait()
    pltpu.make_async_copy(rows[(_NCHT - 1) % _NBUF], acc.at[dstb[(_NCHT - 1) % _NBUF]], ssem[(_NCHT - 1) % _NBUF]).wait()
    plsc.subcore_barrier()

    plsc.subcore_barrier()

    for k in range(_ROWS_PER_TILE // 128):
        sl = pl.ds(rbase + k * 128, 128)

        @pl.when(c == 0)
        def _():
            pltpu.sync_copy(acc.at[sl], out0.at[sl])

        @pl.when(c == 1)
        def _():
            pltpu.sync_copy(acc.at[sl], out1.at[sl])


def _combine_body(pa_ref, pb_ref, run_ref, emb_ref, runo_ref):
    sm = pa_ref[...] + pb_ref[...]
    emb_ref[...] = sm
    runo_ref[...] = run_ref[...] + sm


def _final_body(pa_ref, pb_ref, run_ref, out_ref):
    out_ref[...] = (run_ref[...] + pa_ref[...] + pb_ref[...]) * 0.25


_bs = pl.BlockSpec((1024, _D), lambda i: (i, 0))
_sds = jax.ShapeDtypeStruct((_N_PAD, _D), jnp.float32)

_combine = pl.pallas_call(
    _combine_body, grid=(10,), in_specs=[_bs, _bs, _bs],
    out_specs=[_bs, _bs], out_shape=[_sds, _sds])

_final = pl.pallas_call(
    _final_body, grid=(10,), in_specs=[_bs, _bs, _bs],
    out_specs=_bs, out_shape=_sds)


def kernel(playlist_weight, track_weight, edge_index, edge_weight):
    emb0 = jnp.concatenate([playlist_weight, track_weight], axis=0)
    emb0 = jnp.pad(emb0, ((0, _N_PAD - _N), (0, 0)))
    src = edge_index[0].astype(jnp.int32)
    dst = edge_index[1].astype(jnp.int32)
    w = edge_weight.astype(jnp.float32)
    pad = _E_PAD - _E
    fill = _N + jnp.arange(pad, dtype=jnp.int32) % (_N_PAD - _N)
    src1 = jnp.concatenate([src, fill])
    dst1 = jnp.concatenate([dst, fill])
    w1 = jnp.concatenate([w, jnp.zeros((pad,), jnp.float32)])

    emb = emb0
    run = emb0
    final = None
    for layer in range(_NLAYERS):
        pa, pb = _sc_layer(emb, src1, dst1, w1)
        if layer < _NLAYERS - 1:
            emb, run = _combine(pa, pb, run)
        else:
            final = _final(pa, pb, run)
    return final[:_NUM_PLAYLISTS], final[_NUM_PLAYLISTS:_N]
